# baseline (speedup 1.0000x reference)
import functools

import jax
import jax.numpy as jnp
from jax import lax
from jax.experimental import pallas as pl
from jax.experimental.pallas import tpu as pltpu
from jax.experimental.pallas import tpu_sc as plsc

H = 128
G = 64
NC = 1
NS = 16
NW = NC * NS
CH = 128
NBUF = 2
KG = 16


def _make_spmm(ns, npad, nch):
    rpt = npad // NS
    mesh = plsc.VectorSubcoreMesh(core_axis_name="c", subcore_axis_name="s",
                                  num_cores=NC)

    def body(p_hbm, src_hbm, dst_hbm, z0_hbm, out_hbm,
             srcb, dstb, rowb, acc, sem0, sem1):
        c = lax.axis_index("c")
        s = lax.axis_index("s")
        wid = s * NC + c
        pltpu.sync_copy(z0_hbm, acc.at[pl.ds(s * rpt, rpt)])
        plsc.subcore_barrier()

        def group(kg, carry):
            pltpu.sync_copy(src_hbm.at[wid, pl.ds(kg * KG, KG)], srcb)
            pltpu.sync_copy(dst_hbm.at[wid, pl.ds(kg * KG, KG)], dstb)

            def pair(gp, c2):
                ch0 = gp * 2
                cp0 = pltpu.async_copy(
                    p_hbm.at[srcb.at[ch0]], rowb.at[0], sem0)
                cp1 = pltpu.async_copy(
                    p_hbm.at[srcb.at[ch0 + 1]], rowb.at[1], sem1)
                cp0.wait()
                pltpu.sync_copy(rowb.at[0], acc.at[dstb.at[ch0]], add=True)
                cp1.wait()
                pltpu.sync_copy(rowb.at[1], acc.at[dstb.at[ch0 + 1]],
                                add=True)
                return c2

            lax.fori_loop(0, KG // 2, pair, 0)
            return carry

        lax.fori_loop(0, nch // KG, group, 0)
        plsc.subcore_barrier()
        pltpu.sync_copy(acc.at[pl.ds(s * rpt, rpt)],
                        out_hbm.at[c, pl.ds(s * rpt, rpt)])

    return pl.kernel(
        body,
        out_type=jax.ShapeDtypeStruct((NC, npad, H), jnp.float32),
        mesh=mesh,
        scratch_types=[
            pltpu.VMEM((KG, CH), jnp.int32),
            pltpu.VMEM((KG, CH), jnp.int32),
            pltpu.VMEM((NBUF, CH, H), jnp.float32),
            pltpu.VMEM_SHARED((npad, H), jnp.float32),
            pltpu.SemaphoreType.DMA,
            pltpu.SemaphoreType.DMA,
        ],
    )


def _p0_body(z_ref, x_ref, zt_ref, w0_ref, cc_ref, out_ref):
    maxz = zt_ref.shape[0]
    r = z_ref.shape[0]
    oh = (z_ref[...] == lax.broadcasted_iota(jnp.int32, (r, maxz), 1))
    z_emb = jnp.dot(oh.astype(jnp.float32), zt_ref[...],
                    preferred_element_type=jnp.float32)
    h = jnp.concatenate([z_emb, x_ref[...]], axis=1)
    out_ref[...] = (jnp.dot(h, w0_ref[...],
                            preferred_element_type=jnp.float32)
                    + cc_ref[...])


def _combine_body(p_ref, agg_ref, b_ref, w_ref, out_ref):
    h = p_ref[...] + b_ref[...]
    for i in range(agg_ref.shape[0]):
        h = h + agg_ref[i]
    h = jnp.maximum(h, 0.0)
    out_ref[...] = jnp.dot(h, w_ref[...], preferred_element_type=jnp.float32)


def _pool_body(p_ref, agg_ref, b_ref, bat_ref, wm1_ref, bm1_ref,
               wm2_ref, bm2_ref, out_ref, pool_acc, cnt_acc):
    i = pl.program_id(0)
    nsteps = pl.num_programs(0)
    r = p_ref.shape[0]

    @pl.when(i == 0)
    def _init():
        pool_acc[...] = jnp.zeros_like(pool_acc)
        cnt_acc[...] = jnp.zeros_like(cnt_acc)

    h3 = p_ref[...] + b_ref[...]
    for i in range(agg_ref.shape[0]):
        h3 = h3 + agg_ref[i]
    ohT = (bat_ref[0] == lax.broadcasted_iota(jnp.int32, (G, r), 0))
    ohT = ohT.astype(jnp.float32)
    pool_acc[...] += jnp.dot(ohT, h3, preferred_element_type=jnp.float32)
    cnt_acc[...] += jnp.dot(ohT, jnp.ones((r, H), jnp.float32),
                            preferred_element_type=jnp.float32)

    @pl.when(i == nsteps - 1)
    def _finish():
        pooled = pool_acc[...] / jnp.maximum(cnt_acc[...], 1.0)
        m = jnp.maximum(jnp.dot(pooled, wm1_ref[...],
                                preferred_element_type=jnp.float32)
                        + bm1_ref[...], 0.0)
        out_ref[...] = (jnp.dot(m, wm2_ref[...],
                                preferred_element_type=jnp.float32)
                        + bm2_ref[...])


def kernel(num_nodes, z, edge_index, batch, x, z_table,
           W0, b0, W1, b1, W2, b2, Wm1, bm1, Wm2, bm2):
    ns = z.shape[0]
    e = edge_index.shape[1]
    maxz = z_table.shape[0]
    out_dim = Wm2.shape[1]

    epad = -(-e // (NW * CH * KG)) * (NW * CH * KG)
    nch = epad // (NW * CH)
    npad = -(-(ns + 1) // (NS * 8)) * (NS * 8)

    src = jnp.concatenate(
        [edge_index[0], jnp.zeros((epad - e,), edge_index.dtype)])
    dst = jnp.concatenate(
        [edge_index[1], jnp.full((epad - e,), ns, edge_index.dtype)])
    src = src.reshape(NW, nch, CH).astype(jnp.int32)
    dst = dst.reshape(NW, nch, CH).astype(jnp.int32)
    z0 = jnp.zeros((npad // NS, H), jnp.float32)

    spmm = _make_spmm(ns, npad, nch)

    r0, rb = 1000, 2000
    nb0, nbk = ns // r0, ns // rb

    cc = (jnp.asarray(num_nodes, jnp.float32) - jnp.float32(ns))
    ccvec = (cc * jnp.sum(W0, axis=0)).reshape(1, H)

    p0 = pl.pallas_call(
        _p0_body,
        grid=(nb0,),
        in_specs=[
            pl.BlockSpec((r0, 1), lambda i: (i, 0)),
            pl.BlockSpec((r0, H), lambda i: (i, 0)),
            pl.BlockSpec((maxz, H), lambda i: (0, 0)),
            pl.BlockSpec((2 * H, H), lambda i: (0, 0)),
            pl.BlockSpec((1, H), lambda i: (0, 0)),
        ],
        out_specs=pl.BlockSpec((r0, H), lambda i: (i, 0)),
        out_shape=jax.ShapeDtypeStruct((ns, H), jnp.float32),
    )(z.reshape(ns, 1).astype(jnp.int32), x, z_table, W0, ccvec)

    def combine(p, agg, b, w):
        return pl.pallas_call(
            _combine_body,
            grid=(nbk,),
            in_specs=[
                pl.BlockSpec((rb, H), lambda i: (i, 0)),
                pl.BlockSpec((NC, rb, H), lambda i: (0, i, 0)),
                pl.BlockSpec((1, H), lambda i: (0, 0)),
                pl.BlockSpec((H, H), lambda i: (0, 0)),
            ],
            out_specs=pl.BlockSpec((rb, H), lambda i: (i, 0)),
            out_shape=jax.ShapeDtypeStruct((ns, H), jnp.float32),
        )(p, agg, b.reshape(1, H), w)

    a0 = spmm(p0, src, dst, z0)
    p1 = combine(p0, a0, b0, W1)
    a1 = spmm(p1, src, dst, z0)
    p2 = combine(p1, a1, b1, W2)
    a2 = spmm(p2, src, dst, z0)

    out = pl.pallas_call(
        _pool_body,
        grid=(nbk,),
        in_specs=[
            pl.BlockSpec((rb, H), lambda i: (i, 0)),
            pl.BlockSpec((NC, rb, H), lambda i: (0, i, 0)),
            pl.BlockSpec((1, H), lambda i: (0, 0)),
            pl.BlockSpec((1, 1, rb), lambda i: (i, 0, 0)),
            pl.BlockSpec((H, H), lambda i: (0, 0)),
            pl.BlockSpec((1, H), lambda i: (0, 0)),
            pl.BlockSpec((H, out_dim), lambda i: (0, 0)),
            pl.BlockSpec((1, out_dim), lambda i: (0, 0)),
        ],
        out_specs=pl.BlockSpec((G, out_dim), lambda i: (0, 0)),
        out_shape=jax.ShapeDtypeStruct((G, out_dim), jnp.float32),
        scratch_shapes=[
            pltpu.VMEM((G, H), jnp.float32),
            pltpu.VMEM((G, H), jnp.float32),
        ],
    )(p2, a2, b2.reshape(1, H), batch.reshape(nbk, 1, rb).astype(jnp.int32),
      Wm1, bm1.reshape(1, H), Wm2, bm2.reshape(1, out_dim))

    return out


# speedup vs baseline: 1.1600x; 1.1600x over previous
import functools

import jax
import jax.numpy as jnp
from jax import lax
from jax.experimental import pallas as pl
from jax.experimental.pallas import tpu as pltpu
from jax.experimental.pallas import tpu_sc as plsc

H = 128
G = 64
NC = 2
NS = 16
NW = NC * NS
CH = 128
NBUF = 2
KG = 16


def _make_spmm(ns, npad, nch):
    rpt = npad // NS
    mesh = plsc.VectorSubcoreMesh(core_axis_name="c", subcore_axis_name="s",
                                  num_cores=NC)

    def body(p_hbm, src_hbm, dst_hbm, z0_hbm, out_hbm,
             srcb, dstb, rowb, acc, sem0, sem1):
        c = lax.axis_index("c")
        s = lax.axis_index("s")
        wid = s * NC + c
        pltpu.sync_copy(z0_hbm, acc.at[pl.ds(s * rpt, rpt)])
        plsc.subcore_barrier()

        def group(kg, carry):
            pltpu.sync_copy(src_hbm.at[wid, pl.ds(kg * KG, KG)], srcb)
            pltpu.sync_copy(dst_hbm.at[wid, pl.ds(kg * KG, KG)], dstb)

            def pair(gp, c2):
                ch0 = gp * 2
                cp0 = pltpu.async_copy(
                    p_hbm.at[srcb.at[ch0]], rowb.at[0], sem0)
                cp1 = pltpu.async_copy(
                    p_hbm.at[srcb.at[ch0 + 1]], rowb.at[1], sem1)
                cp0.wait()
                pltpu.sync_copy(rowb.at[0], acc.at[dstb.at[ch0]], add=True)
                cp1.wait()
                pltpu.sync_copy(rowb.at[1], acc.at[dstb.at[ch0 + 1]],
                                add=True)
                return c2

            lax.fori_loop(0, KG // 2, pair, 0)
            return carry

        lax.fori_loop(0, nch // KG, group, 0)
        plsc.subcore_barrier()
        pltpu.sync_copy(acc.at[pl.ds(s * rpt, rpt)],
                        out_hbm.at[c, pl.ds(s * rpt, rpt)])

    return pl.kernel(
        body,
        out_type=jax.ShapeDtypeStruct((NC, npad, H), jnp.float32),
        mesh=mesh,
        scratch_types=[
            pltpu.VMEM((KG, CH), jnp.int32),
            pltpu.VMEM((KG, CH), jnp.int32),
            pltpu.VMEM((NBUF, CH, H), jnp.float32),
            pltpu.VMEM_SHARED((npad, H), jnp.float32),
            pltpu.SemaphoreType.DMA,
            pltpu.SemaphoreType.DMA,
        ],
    )


def _p0_body(z_ref, x_ref, zt_ref, w0_ref, cc_ref, out_ref):
    maxz = zt_ref.shape[0]
    r = z_ref.shape[0]
    oh = (z_ref[...] == lax.broadcasted_iota(jnp.int32, (r, maxz), 1))
    z_emb = jnp.dot(oh.astype(jnp.float32), zt_ref[...],
                    preferred_element_type=jnp.float32)
    h = jnp.concatenate([z_emb, x_ref[...]], axis=1)
    out_ref[...] = (jnp.dot(h, w0_ref[...],
                            preferred_element_type=jnp.float32)
                    + cc_ref[...])


def _combine_body(p_ref, agg_ref, b_ref, w_ref, out_ref):
    h = p_ref[...] + b_ref[...]
    for i in range(agg_ref.shape[0]):
        h = h + agg_ref[i]
    h = jnp.maximum(h, 0.0)
    out_ref[...] = jnp.dot(h, w_ref[...], preferred_element_type=jnp.float32)


def _pool_body(p_ref, agg_ref, b_ref, bat_ref, wm1_ref, bm1_ref,
               wm2_ref, bm2_ref, out_ref, pool_acc, cnt_acc):
    i = pl.program_id(0)
    nsteps = pl.num_programs(0)
    r = p_ref.shape[0]

    @pl.when(i == 0)
    def _init():
        pool_acc[...] = jnp.zeros_like(pool_acc)
        cnt_acc[...] = jnp.zeros_like(cnt_acc)

    h3 = p_ref[...] + b_ref[...]
    for i in range(agg_ref.shape[0]):
        h3 = h3 + agg_ref[i]
    ohT = (bat_ref[0] == lax.broadcasted_iota(jnp.int32, (G, r), 0))
    ohT = ohT.astype(jnp.float32)
    pool_acc[...] += jnp.dot(ohT, h3, preferred_element_type=jnp.float32)
    cnt_acc[...] += jnp.dot(ohT, jnp.ones((r, H), jnp.float32),
                            preferred_element_type=jnp.float32)

    @pl.when(i == nsteps - 1)
    def _finish():
        pooled = pool_acc[...] / jnp.maximum(cnt_acc[...], 1.0)
        m = jnp.maximum(jnp.dot(pooled, wm1_ref[...],
                                preferred_element_type=jnp.float32)
                        + bm1_ref[...], 0.0)
        out_ref[...] = (jnp.dot(m, wm2_ref[...],
                                preferred_element_type=jnp.float32)
                        + bm2_ref[...])


def kernel(num_nodes, z, edge_index, batch, x, z_table,
           W0, b0, W1, b1, W2, b2, Wm1, bm1, Wm2, bm2):
    ns = z.shape[0]
    e = edge_index.shape[1]
    maxz = z_table.shape[0]
    out_dim = Wm2.shape[1]

    epad = -(-e // (NW * CH * KG)) * (NW * CH * KG)
    nch = epad // (NW * CH)
    npad = -(-(ns + 1) // (NS * 8)) * (NS * 8)

    src = jnp.concatenate(
        [edge_index[0], jnp.zeros((epad - e,), edge_index.dtype)])
    dst = jnp.concatenate(
        [edge_index[1], jnp.full((epad - e,), ns, edge_index.dtype)])
    src = src.reshape(NW, nch, CH).astype(jnp.int32)
    dst = dst.reshape(NW, nch, CH).astype(jnp.int32)
    coff = (jnp.arange(NW, dtype=jnp.int32) % NC * ns).reshape(NW, 1, 1)
    src = src + coff
    z0 = jnp.zeros((npad // NS, H), jnp.float32)

    spmm = _make_spmm(ns, npad, nch)

    r0, rb = 1000, 2000
    nb0, nbk = ns // r0, ns // rb

    cc = (jnp.asarray(num_nodes, jnp.float32) - jnp.float32(ns))
    ccvec = (cc * jnp.sum(W0, axis=0)).reshape(1, H)

    p0 = pl.pallas_call(
        _p0_body,
        grid=(nb0,),
        in_specs=[
            pl.BlockSpec((r0, 1), lambda i: (i, 0)),
            pl.BlockSpec((r0, H), lambda i: (i, 0)),
            pl.BlockSpec((maxz, H), lambda i: (0, 0)),
            pl.BlockSpec((2 * H, H), lambda i: (0, 0)),
            pl.BlockSpec((1, H), lambda i: (0, 0)),
        ],
        out_specs=pl.BlockSpec((r0, H), lambda i: (i, 0)),
        out_shape=jax.ShapeDtypeStruct((ns, H), jnp.float32),
    )(z.reshape(ns, 1).astype(jnp.int32), x, z_table, W0, ccvec)

    def combine(p, agg, b, w):
        return pl.pallas_call(
            _combine_body,
            grid=(nbk,),
            in_specs=[
                pl.BlockSpec((rb, H), lambda i: (i, 0)),
                pl.BlockSpec((NC, rb, H), lambda i: (0, i, 0)),
                pl.BlockSpec((1, H), lambda i: (0, 0)),
                pl.BlockSpec((H, H), lambda i: (0, 0)),
            ],
            out_specs=pl.BlockSpec((rb, H), lambda i: (i, 0)),
            out_shape=jax.ShapeDtypeStruct((ns, H), jnp.float32),
        )(p, agg, b.reshape(1, H), w)

    def dup(p):
        return jnp.concatenate([p, p], axis=0)

    a0 = spmm(dup(p0), src, dst, z0)
    p1 = combine(p0, a0, b0, W1)
    a1 = spmm(dup(p1), src, dst, z0)
    p2 = combine(p1, a1, b1, W2)
    a2 = spmm(dup(p2), src, dst, z0)

    out = pl.pallas_call(
        _pool_body,
        grid=(nbk,),
        in_specs=[
            pl.BlockSpec((rb, H), lambda i: (i, 0)),
            pl.BlockSpec((NC, rb, H), lambda i: (0, i, 0)),
            pl.BlockSpec((1, H), lambda i: (0, 0)),
            pl.BlockSpec((1, 1, rb), lambda i: (i, 0, 0)),
            pl.BlockSpec((H, H), lambda i: (0, 0)),
            pl.BlockSpec((1, H), lambda i: (0, 0)),
            pl.BlockSpec((H, out_dim), lambda i: (0, 0)),
            pl.BlockSpec((1, out_dim), lambda i: (0, 0)),
        ],
        out_specs=pl.BlockSpec((G, out_dim), lambda i: (0, 0)),
        out_shape=jax.ShapeDtypeStruct((G, out_dim), jnp.float32),
        scratch_shapes=[
            pltpu.VMEM((G, H), jnp.float32),
            pltpu.VMEM((G, H), jnp.float32),
        ],
    )(p2, a2, b2.reshape(1, H), batch.reshape(nbk, 1, rb).astype(jnp.int32),
      Wm1, bm1.reshape(1, H), Wm2, bm2.reshape(1, out_dim))

    return out
